# baseline (speedup 1.0000x reference)
import functools

import jax
import jax.numpy as jnp
from jax import lax
from jax.experimental import pallas as pl
from jax.experimental.pallas import tpu as pltpu
from jax.experimental.pallas import tpu_sc as plsc

N = 10000
E = 320000
HEADS = 8
DIM = 16
D = HEADS * DIM
HALF = D // 2

NC = 2
NS = 16
NW = NC * NS

C = 112
NBUF = 5
NCHUNK = 180
PER_TILE = NCHUNK * C
E_PAD = PER_TILE * NS
NMETA = 6

STRIPE = 632
STRIPE_LAST = N - 15 * STRIPE


def _mm_body(x_ref, w_ref, o_ref):
    o_ref[...] = jnp.dot(x_ref[...], w_ref[0],
                         preferred_element_type=jnp.float32)[None]


def _project_split(x, W):
    w_split = W.reshape(D, NC, HALF).transpose(1, 0, 2)
    return pl.pallas_call(
        _mm_body,
        grid=(10, NC),
        in_specs=[
            pl.BlockSpec((N // 10, D), lambda i, j: (i, 0)),
            pl.BlockSpec((1, D, HALF), lambda i, j: (j, 0, 0)),
        ],
        out_specs=pl.BlockSpec((1, N // 10, HALF), lambda i, j: (j, i, 0)),
        out_shape=jax.ShapeDtypeStruct((NC, N, HALF), jnp.float32),
    )(x, w_split)


def _combine(partials, bias):
    bias2d = jnp.broadcast_to(bias.reshape(1, D), (8, D))

    def body(p_ref, b_ref, o_ref):
        o_ref[...] = (
            jnp.concatenate([p_ref[0], p_ref[1]], axis=-1) + b_ref[0:1, :])

    return pl.pallas_call(
        body,
        grid=(10,),
        in_specs=[
            pl.BlockSpec((NC, N // 10, HALF), lambda i: (0, i, 0)),
            pl.BlockSpec((8, D), lambda i: (0, 0)),
        ],
        out_specs=pl.BlockSpec((N // 10, D), lambda i: (i, 0)),
        out_shape=jax.ShapeDtypeStruct((N, D), jnp.float32),
    )(partials, bias2d)


def _sc_body(h_hbm, meta_hbm, zeros_hbm, out_hbm,
             meta, rows, shared, gsem, ssem, isem):
    hsp = shared.at[0]
    acc = shared.at[1]
    c_id = lax.axis_index("c")
    s_id = lax.axis_index("s")

    start = pl.multiple_of(s_id * STRIPE, 8)

    @pl.when(s_id < NS - 1)
    def _stage_main():
        sl = pl.ds(start, STRIPE)
        pltpu.sync_copy(h_hbm.at[c_id].at[sl], hsp.at[sl])
        pltpu.sync_copy(zeros_hbm.at[sl], acc.at[sl])

    @pl.when(s_id == NS - 1)
    def _stage_last():
        sl = pl.ds((NS - 1) * STRIPE, STRIPE_LAST)
        pltpu.sync_copy(h_hbm.at[c_id].at[sl], hsp.at[sl])
        pltpu.sync_copy(zeros_hbm.at[sl], acc.at[sl])

    plsc.subcore_barrier()

    chunk0 = s_id * NCHUNK


    def i_desc(j, p):
        return pltpu.make_async_copy(
            meta_hbm.at[c_id].at[chunk0 + j], meta.at[p], isem.at[p])

    def g_desc(p):
        return pltpu.make_async_copy(
            hsp.at[meta.at[p, 0]], rows.at[p], gsem.at[p])

    def s_desc(p):
        return pltpu.make_async_copy(
            rows.at[p], acc.at[meta.at[p, 1]], ssem.at[p])

    def multiply(p):
        nh = HEADS // NC

        def group(q, carry2):
            r = 2 + q // (C // 16)
            col = 16 * (q % (C // 16))
            av = lax.bitcast_convert_type(
                meta[p, r, pl.ds(col, 16)], jnp.float32)
            for e in range(4):
                c = 4 * q + e
                for hd in range(nh):
                    rows[p, c, pl.ds(hd * DIM, DIM)] = (
                        rows[p, c, pl.ds(hd * DIM, DIM)] * av[nh * e + hd])
            return carry2

        lax.fori_loop(0, C // 4, group, 0, unroll=2)

    for jj in range(4):
        i_desc(jj, jj).start()
    for jj in range(3):
        i_desc(jj, jj).wait()
        g_desc(jj).start()

    def body(t, carry):
        for p in range(NBUF):
            j = NBUF * t + p
            p3 = (p + 3) % NBUF
            p4 = (p + 4) % NBUF

            @pl.when(j >= 1)
            def _wait_s():
                s_desc(p4).wait()

            @pl.when(j + 4 < NCHUNK)
            def _issue_i():
                i_desc(j + 4, p4).start()

            g_desc(p).wait()
            multiply(p)
            pltpu.async_copy(
                rows.at[p], acc.at[meta.at[p, 1]], ssem.at[p], add=True)

            @pl.when(j + 3 < NCHUNK)
            def _issue_g():
                i_desc(j + 3, p3).wait()
                g_desc(p3).start()
        return carry

    lax.fori_loop(0, NCHUNK // NBUF, body, 0)
    s_desc((NCHUNK - 1) % NBUF).wait()
    plsc.subcore_barrier()

    @pl.when(s_id < NS - 1)
    def _flush_main():
        sl = pl.ds(start, STRIPE)
        pltpu.sync_copy(acc.at[sl], out_hbm.at[c_id].at[sl])

    @pl.when(s_id == NS - 1)
    def _flush_last():
        sl = pl.ds((NS - 1) * STRIPE, STRIPE_LAST)
        pltpu.sync_copy(acc.at[sl], out_hbm.at[c_id].at[sl])


@functools.partial(jax.jit, static_argnums=())
def _sc_scatter(h, meta, zeros):
    mesh = plsc.VectorSubcoreMesh(core_axis_name="c", subcore_axis_name="s",
                                  num_cores=NC, num_subcores=NS)
    f = pl.kernel(
        _sc_body,
        out_type=jax.ShapeDtypeStruct((NC, N, HALF), jnp.float32),
        mesh=mesh,
        compiler_params=pltpu.CompilerParams(use_tc_tiling_on_sc=False),
        scratch_types=[
            pltpu.VMEM((NBUF, NMETA, C), jnp.int32),
            pltpu.VMEM((NBUF, C, HALF), jnp.float32),
            pltpu.VMEM_SHARED((NC, N, HALF), jnp.float32),
            pltpu.SemaphoreType.DMA((NBUF,)),
            pltpu.SemaphoreType.DMA((NBUF,)),
            pltpu.SemaphoreType.DMA((NBUF,)),
        ],
    )
    return f(h, meta, zeros)


def kernel(x, edge_index, agg_alpha, W, bias):
    h = _project_split(x, W)
    pad = E_PAD - E
    tot = E_PAD // C
    src = jnp.concatenate([edge_index[0], jnp.zeros((pad,), jnp.int32)])
    dst = jnp.concatenate([edge_index[1], jnp.zeros((pad,), jnp.int32)])
    alpha = jnp.concatenate(
        [agg_alpha, jnp.zeros((pad, HEADS), jnp.float32)], axis=0)
    nh = HEADS // NC
    metas = []
    for cc in range(NC):
        metas.append(jnp.concatenate(
            [src.reshape(tot, 1, C),
             dst.reshape(tot, 1, C),
             lax.bitcast_convert_type(
                 alpha[:, cc * nh:(cc + 1) * nh],
                 jnp.int32).reshape(tot, nh, C)],
            axis=1))
    meta = jnp.stack(metas, axis=0)
    zeros = jnp.zeros((N, HALF), jnp.float32)
    partials = _sc_scatter(h, meta, zeros)
    return _combine(partials, bias)


# speedup vs baseline: 1.0946x; 1.0946x over previous
import functools

import jax
import jax.numpy as jnp
from jax import lax
from jax.experimental import pallas as pl
from jax.experimental.pallas import tpu as pltpu
from jax.experimental.pallas import tpu_sc as plsc

N = 10000
E = 320000
HEADS = 8
DIM = 16
D = HEADS * DIM
HALF = D // 2

NC = 2
NS = 16
NW = NC * NS

C = 112
NBUF = 4
NCHUNK = 180
PER_TILE = NCHUNK * C
E_PAD = PER_TILE * NS
NMETA = 6

STRIPE = 632
STRIPE_LAST = N - 15 * STRIPE


def _mm_body(x_ref, w_ref, o_ref):
    o_ref[...] = jnp.dot(x_ref[...], w_ref[0],
                         preferred_element_type=jnp.float32)[None]


def _project_split(x, W):
    w_split = W.reshape(D, NC, HALF).transpose(1, 0, 2)
    return pl.pallas_call(
        _mm_body,
        grid=(10, NC),
        in_specs=[
            pl.BlockSpec((N // 10, D), lambda i, j: (i, 0)),
            pl.BlockSpec((1, D, HALF), lambda i, j: (j, 0, 0)),
        ],
        out_specs=pl.BlockSpec((1, N // 10, HALF), lambda i, j: (j, i, 0)),
        out_shape=jax.ShapeDtypeStruct((NC, N, HALF), jnp.float32),
    )(x, w_split)


def _combine(partials, bias):
    bias2d = jnp.broadcast_to(bias.reshape(1, D), (8, D))

    def body(p_ref, b_ref, o_ref):
        o_ref[...] = (
            jnp.concatenate([p_ref[0], p_ref[1]], axis=-1) + b_ref[0:1, :])

    return pl.pallas_call(
        body,
        grid=(10,),
        in_specs=[
            pl.BlockSpec((NC, N // 10, HALF), lambda i: (0, i, 0)),
            pl.BlockSpec((8, D), lambda i: (0, 0)),
        ],
        out_specs=pl.BlockSpec((N // 10, D), lambda i: (i, 0)),
        out_shape=jax.ShapeDtypeStruct((N, D), jnp.float32),
    )(partials, bias2d)


def _sc_body(h_hbm, meta_hbm, zeros_hbm, out_hbm,
             meta, rows, shared, gsem, ssem, isem):
    hsp = shared.at[0]
    acc = shared.at[1]
    c_id = lax.axis_index("c")
    s_id = lax.axis_index("s")

    start = pl.multiple_of(s_id * STRIPE, 8)

    @pl.when(s_id < NS - 1)
    def _stage_main():
        sl = pl.ds(start, STRIPE)
        pltpu.sync_copy(h_hbm.at[c_id].at[sl], hsp.at[sl])
        pltpu.sync_copy(zeros_hbm.at[sl], acc.at[sl])

    @pl.when(s_id == NS - 1)
    def _stage_last():
        sl = pl.ds((NS - 1) * STRIPE, STRIPE_LAST)
        pltpu.sync_copy(h_hbm.at[c_id].at[sl], hsp.at[sl])
        pltpu.sync_copy(zeros_hbm.at[sl], acc.at[sl])

    plsc.subcore_barrier()

    chunk0 = s_id * NCHUNK


    def i_desc(j, p):
        return pltpu.make_async_copy(
            meta_hbm.at[c_id].at[chunk0 + j], meta.at[p], isem.at[p])

    def g_desc(p):
        return pltpu.make_async_copy(
            hsp.at[meta.at[p, 0]], rows.at[p], gsem.at[p])

    def s_desc(p):
        return pltpu.make_async_copy(
            rows.at[p], acc.at[meta.at[p, 1]], ssem.at[p])

    def multiply(p):
        nh = HEADS // NC

        def group(q, carry2):
            r = 2 + q // (C // 16)
            col = 16 * (q % (C // 16))
            av = lax.bitcast_convert_type(
                meta[p, r, pl.ds(col, 16)], jnp.float32)
            for e in range(4):
                c = 4 * q + e
                for hd in range(nh):
                    rows[p, c, pl.ds(hd * DIM, DIM)] = (
                        rows[p, c, pl.ds(hd * DIM, DIM)] * av[nh * e + hd])
            return carry2

        lax.fori_loop(0, C // 4, group, 0, unroll=2)

    for jj in range(2):
        i_desc(jj, jj).start()
    for jj in range(2):
        i_desc(jj, jj).wait()
        g_desc(jj).start()

    def body(t, carry):
        for p in range(NBUF):
            j = NBUF * t + p
            p2 = (p + 2) % NBUF

            @pl.when(j >= NBUF - 2)
            def _wait_s():
                s_desc(p2).wait()

            @pl.when(j + 2 < NCHUNK)
            def _issue_i():
                i_desc(j + 2, p2).start()

            g_desc(p).wait()
            multiply(p)
            pltpu.async_copy(
                rows.at[p], acc.at[meta.at[p, 1]], ssem.at[p], add=True)

            @pl.when(j + 2 < NCHUNK)
            def _issue_g():
                i_desc(j + 2, p2).wait()
                g_desc(p2).start()
        return carry

    lax.fori_loop(0, NCHUNK // NBUF, body, 0)
    for k in range(NBUF - 2):
        s_desc((NCHUNK - (NBUF - 2) + k) % NBUF).wait()
    plsc.subcore_barrier()

    @pl.when(s_id < NS - 1)
    def _flush_main():
        sl = pl.ds(start, STRIPE)
        pltpu.sync_copy(acc.at[sl], out_hbm.at[c_id].at[sl])

    @pl.when(s_id == NS - 1)
    def _flush_last():
        sl = pl.ds((NS - 1) * STRIPE, STRIPE_LAST)
        pltpu.sync_copy(acc.at[sl], out_hbm.at[c_id].at[sl])


@functools.partial(jax.jit, static_argnums=())
def _sc_scatter(h, meta, zeros):
    mesh = plsc.VectorSubcoreMesh(core_axis_name="c", subcore_axis_name="s",
                                  num_cores=NC, num_subcores=NS)
    f = pl.kernel(
        _sc_body,
        out_type=jax.ShapeDtypeStruct((NC, N, HALF), jnp.float32),
        mesh=mesh,
        compiler_params=pltpu.CompilerParams(use_tc_tiling_on_sc=False),
        scratch_types=[
            pltpu.VMEM((NBUF, NMETA, C), jnp.int32),
            pltpu.VMEM((NBUF, C, HALF), jnp.float32),
            pltpu.VMEM_SHARED((NC, N, HALF), jnp.float32),
            pltpu.SemaphoreType.DMA((NBUF,)),
            pltpu.SemaphoreType.DMA((NBUF,)),
            pltpu.SemaphoreType.DMA((NBUF,)),
        ],
    )
    return f(h, meta, zeros)


def kernel(x, edge_index, agg_alpha, W, bias):
    h = _project_split(x, W)
    pad = E_PAD - E
    tot = E_PAD // C
    src = jnp.concatenate([edge_index[0], jnp.zeros((pad,), jnp.int32)])
    dst = jnp.concatenate([edge_index[1], jnp.zeros((pad,), jnp.int32)])
    alpha = jnp.concatenate(
        [agg_alpha, jnp.zeros((pad, HEADS), jnp.float32)], axis=0)
    nh = HEADS // NC
    metas = []
    for cc in range(NC):
        metas.append(jnp.concatenate(
            [src.reshape(tot, 1, C),
             dst.reshape(tot, 1, C),
             lax.bitcast_convert_type(
                 alpha[:, cc * nh:(cc + 1) * nh],
                 jnp.int32).reshape(tot, nh, C)],
            axis=1))
    meta = jnp.stack(metas, axis=0)
    zeros = jnp.zeros((N, HALF), jnp.float32)
    partials = _sc_scatter(h, meta, zeros)
    return _combine(partials, bias)
